# Initial kernel scaffold; baseline (speedup 1.0000x reference)
import jax
import jax.numpy as jnp
from jax.experimental import pallas as pl


def kernel(x, edge_index, W1, att_src, att_dst, W2):
    raise NotImplementedError("write your pallas kernel here")


# speedup vs baseline: 17.3093x; 17.3093x over previous
import functools

import jax
import jax.numpy as jnp
from jax import lax
from jax.experimental import pallas as pl
from jax.experimental.pallas import tpu as pltpu
from jax.experimental.pallas import tpu_sc as plsc

N = 10000
E = 320000
D_IN = 128
D_HID = 512
D_LAT = 30

NPAD = 10240
CHUNK = 80
NC, NS = 2, 16
EPW = E // (NC * NS)
NCHUNK = EPW // CHUNK
RPS = NPAD // NS


def _sigmoid(v):
    return 1.0 / (1.0 + jnp.exp(-v))


def _sc_pass1_body(x_hbm, asrc_hbm, adst_hbm, src_hbm, dst_hbm, zbig_hbm,
                   zvec_hbm, g1p_hbm, sp_hbm, p_hbm,
                   idx_s, idx_d, rows, av, dv, pb, g1_sh, s_sh, sem_g, sem_r):
    c = lax.axis_index("c")
    sub = lax.axis_index("s")
    wid = sub * NC + c
    pltpu.sync_copy(zbig_hbm, g1_sh.at[pl.ds(sub * RPS, RPS)])
    pltpu.sync_copy(zvec_hbm, s_sh.at[pl.ds(sub * RPS, RPS)])
    plsc.subcore_barrier()

    base = wid * EPW

    def chunk(k, carry):
        off = base + k * CHUNK
        pltpu.sync_copy(src_hbm.at[pl.ds(off, CHUNK)], idx_s)
        pltpu.sync_copy(dst_hbm.at[pl.ds(off, CHUNK)], idx_d)
        pltpu.async_copy(asrc_hbm.at[idx_s], av, sem_g).wait()
        pltpu.async_copy(adst_hbm.at[idx_d], dv, sem_g).wait()
        pltpu.async_copy(x_hbm.at[idx_s], rows, sem_r).wait()
        for i in range(CHUNK // 16):
            sl = pl.ds(i * 16, 16)
            pb[sl] = jnp.exp(_sigmoid(av[sl] + dv[sl]))
        pltpu.sync_copy(pb, p_hbm.at[pl.ds(off, CHUNK)])
        pltpu.sync_copy(pb, s_sh.at[idx_d], add=True)
        for i in range(CHUNK // 16):
            pbv = pb[pl.ds(i * 16, 16)]
            for j in range(16):
                e = i * 16 + j
                pv = jnp.take_along_axis(
                    pbv, jnp.full((16,), j, jnp.int32), axis=0,
                    mode="promise_in_bounds")
                for k in range(D_IN // 16):
                    sl = pl.ds(k * 16, 16)
                    rows[e, sl] = rows[e, sl] * pv
        pltpu.sync_copy(rows, g1_sh.at[idx_d], add=True)
        return carry

    lax.fori_loop(0, NCHUNK, chunk, 0)
    plsc.subcore_barrier()
    pltpu.sync_copy(g1_sh.at[pl.ds(sub * RPS, RPS)],
                    g1p_hbm.at[c, pl.ds(sub * RPS, RPS)])
    pltpu.sync_copy(s_sh.at[pl.ds(sub * RPS, RPS)],
                    sp_hbm.at[c, pl.ds(sub * RPS, RPS)])


def _sc_pass3_body(x2_hbm, p_hbm, src_hbm, dst_hbm, zsm_hbm, g3p_hbm,
                   idx_s, idx_d, rows, pb, g3_sh, sem_r):
    c = lax.axis_index("c")
    sub = lax.axis_index("s")
    wid = sub * NC + c
    pltpu.sync_copy(zsm_hbm, g3_sh.at[pl.ds(sub * RPS, RPS)])
    plsc.subcore_barrier()

    base = wid * EPW

    def chunk(k, carry):
        off = base + k * CHUNK
        pltpu.sync_copy(src_hbm.at[pl.ds(off, CHUNK)], idx_s)
        pltpu.sync_copy(dst_hbm.at[pl.ds(off, CHUNK)], idx_d)
        pltpu.sync_copy(p_hbm.at[pl.ds(off, CHUNK)], pb)
        pltpu.async_copy(x2_hbm.at[idx_s], rows, sem_r).wait()
        for i in range(CHUNK // 16):
            pbv = pb[pl.ds(i * 16, 16)]
            for j in range(16):
                e = i * 16 + j
                pv = jnp.take_along_axis(
                    pbv, jnp.full((16,), j, jnp.int32), axis=0,
                    mode="promise_in_bounds")
                for k in range(2):
                    sl = pl.ds(k * 16, 16)
                    rows[e, sl] = rows[e, sl] * pv
        pltpu.sync_copy(rows, g3_sh.at[idx_d], add=True)
        return carry

    lax.fori_loop(0, NCHUNK, chunk, 0)
    plsc.subcore_barrier()
    pltpu.sync_copy(g3_sh.at[pl.ds(sub * RPS, RPS)],
                    g3p_hbm.at[c, pl.ds(sub * RPS, RPS)])


def _tc_attn_body(x_ref, w1_ref, att2_ref, a2_ref):
    w12 = lax.dot_general(w1_ref[...], att2_ref[...], (((1,), (0,)), ((), ())),
                          preferred_element_type=jnp.float32)
    a2_ref[...] = lax.dot_general(w12, x_ref[...], (((0,), (1,)), ((), ())),
                                  preferred_element_type=jnp.float32)


def _elu(v):
    return jnp.where(v > 0.0, v, jnp.exp(jnp.minimum(v, 0.0)) - 1.0)


def _tc_mid_body(g1p_ref, sp_ref, w1_ref, w2_ref, x2_ref, x2p_ref, *, blk):
    s = sp_ref[:, 0] + sp_ref[:, 1] + 1e-16
    g1 = (g1p_ref[0] + g1p_ref[1]) / s[:, None]
    x1 = _elu(jnp.dot(g1, w1_ref[...], preferred_element_type=jnp.float32))
    x2 = jnp.dot(x1, w2_ref[...], preferred_element_type=jnp.float32)
    x2_ref[...] = x2
    x2p_ref[...] = jnp.concatenate(
        [x2, jnp.zeros((blk, 32 - D_LAT), jnp.float32)], axis=1)


def _tc_final_body(g3p_ref, sp_ref, w1_ref, w2_ref, x4_ref, *, blk):
    s = sp_ref[:, 0] + sp_ref[:, 1] + 1e-16
    g3 = (g3p_ref[0] + g3p_ref[1])[:, :D_LAT] / s[:, None]
    x3 = _elu(lax.dot_general(g3, w2_ref[...], (((1,), (1,)), ((), ())),
                              preferred_element_type=jnp.float32))
    x4_ref[...] = lax.dot_general(x3, w1_ref[...], (((1,), (1,)), ((), ())),
                                  preferred_element_type=jnp.float32)


@jax.jit
def kernel(x, edge_index, W1, att_src, att_dst, W2):
    src = edge_index[0]
    dst = edge_index[1]
    att2 = jnp.stack([att_src, att_dst], axis=1)

    a2 = pl.pallas_call(
        _tc_attn_body,
        out_shape=jax.ShapeDtypeStruct((2, N), jnp.float32),
    )(x, W1, att2)

    mesh = plsc.VectorSubcoreMesh(core_axis_name="c", subcore_axis_name="s")
    sc_params = pltpu.CompilerParams(needs_layout_passes=False,
                                     use_tc_tiling_on_sc=False)
    zbig = jnp.zeros((RPS, D_IN), jnp.float32)
    zvec = jnp.zeros((RPS,), jnp.float32)
    zsm = jnp.zeros((RPS, 32), jnp.float32)

    sc1 = pl.kernel(
        _sc_pass1_body,
        out_type=[
            jax.ShapeDtypeStruct((NC, NPAD, D_IN), jnp.float32),
            jax.ShapeDtypeStruct((NC, NPAD), jnp.float32),
            jax.ShapeDtypeStruct((E,), jnp.float32),
        ],
        mesh=mesh,
        scratch_types=[
            pltpu.VMEM((CHUNK,), jnp.int32),
            pltpu.VMEM((CHUNK,), jnp.int32),
            pltpu.VMEM((CHUNK, D_IN), jnp.float32),
            pltpu.VMEM((CHUNK,), jnp.float32),
            pltpu.VMEM((CHUNK,), jnp.float32),
            pltpu.VMEM((CHUNK,), jnp.float32),
            pltpu.VMEM_SHARED((NPAD, D_IN), jnp.float32),
            pltpu.VMEM_SHARED((NPAD,), jnp.float32),
            pltpu.SemaphoreType.DMA,
            pltpu.SemaphoreType.DMA,
        ],
        compiler_params=sc_params,
    )
    g1p, sp, p = sc1(x, a2[0], a2[1], src, dst, zbig, zvec)
    sp_t = sp.T

    blk = 1000
    x2, x2p = pl.pallas_call(
        functools.partial(_tc_mid_body, blk=blk),
        grid=(N // blk,),
        in_specs=[
            pl.BlockSpec((NC, blk, D_IN), lambda i: (0, i, 0)),
            pl.BlockSpec((blk, NC), lambda i: (i, 0)),
            pl.BlockSpec((D_IN, D_HID), lambda i: (0, 0)),
            pl.BlockSpec((D_HID, D_LAT), lambda i: (0, 0)),
        ],
        out_specs=[
            pl.BlockSpec((blk, D_LAT), lambda i: (i, 0)),
            pl.BlockSpec((blk, 32), lambda i: (i, 0)),
        ],
        out_shape=[
            jax.ShapeDtypeStruct((N, D_LAT), jnp.float32),
            jax.ShapeDtypeStruct((N, 32), jnp.float32),
        ],
    )(g1p, sp_t, W1, W2)

    sc3 = pl.kernel(
        _sc_pass3_body,
        out_type=jax.ShapeDtypeStruct((NC, NPAD, 32), jnp.float32),
        mesh=mesh,
        scratch_types=[
            pltpu.VMEM((CHUNK,), jnp.int32),
            pltpu.VMEM((CHUNK,), jnp.int32),
            pltpu.VMEM((CHUNK, 32), jnp.float32),
            pltpu.VMEM((CHUNK,), jnp.float32),
            pltpu.VMEM_SHARED((NPAD, 32), jnp.float32),
            pltpu.SemaphoreType.DMA,
        ],
        compiler_params=sc_params,
    )
    g3p = sc3(x2p, p, src, dst, zsm)

    x4 = pl.pallas_call(
        functools.partial(_tc_final_body, blk=blk),
        grid=(N // blk,),
        in_specs=[
            pl.BlockSpec((NC, blk, 32), lambda i: (0, i, 0)),
            pl.BlockSpec((blk, NC), lambda i: (i, 0)),
            pl.BlockSpec((D_IN, D_HID), lambda i: (0, 0)),
            pl.BlockSpec((D_HID, D_LAT), lambda i: (0, 0)),
        ],
        out_specs=pl.BlockSpec((blk, D_IN), lambda i: (i, 0)),
        out_shape=jax.ShapeDtypeStruct((N, D_IN), jnp.float32),
    )(g3p, sp_t, W1, W2)

    return (x2, x4)


# speedup vs baseline: 39.0486x; 2.2559x over previous
import functools

import jax
import jax.numpy as jnp
from jax import lax
from jax.experimental import pallas as pl
from jax.experimental.pallas import tpu as pltpu
from jax.experimental.pallas import tpu_sc as plsc

N = 10000
E = 320000
D_IN = 128
D_HID = 512
D_LAT = 30

NPAD = 10240
CHUNK = 80
NC, NS = 2, 16
EPW = E // (NC * NS)
NCHUNK = EPW // CHUNK
RPS = NPAD // NS


def _sigmoid(v):
    return 1.0 / (1.0 + jnp.exp(-v))


def _scale_rows(pb, rows, dvregs):
    for i in range(CHUNK // 16):
        pbv = pb[pl.ds(i * 16, 16)]
        for j in range(16):
            e = i * 16 + j
            pv = jnp.take_along_axis(
                pbv, jnp.full((16,), j, jnp.int32), axis=0,
                mode="promise_in_bounds")
            for k in range(dvregs):
                sl = pl.ds(k * 16, 16)
                rows[e, sl] = rows[e, sl] * pv


def _sc_pass1_body(x_hbm, asrc_hbm, adst_hbm, src2_hbm, dst2_hbm, zbig_hbm,
                   zvec_hbm, g1p_hbm, sp_hbm, p_hbm,
                   src2d, dst2d, rows0, av0, dv0, pb0, rows1, av1, dv1, pb1,
                   g1_sh, s_sh, sem_g0, sem_g1):
    c = lax.axis_index("c")
    sub = lax.axis_index("s")
    wid = sub * NC + c
    pltpu.sync_copy(zbig_hbm, g1_sh.at[pl.ds(sub * RPS, RPS)])
    pltpu.sync_copy(zvec_hbm, s_sh.at[pl.ds(sub * RPS, RPS)])
    pltpu.sync_copy(src2_hbm.at[pl.ds(wid * NCHUNK, NCHUNK)], src2d)
    pltpu.sync_copy(dst2_hbm.at[pl.ds(wid * NCHUNK, NCHUNK)], dst2d)
    plsc.subcore_barrier()

    base = wid * EPW
    RW = (rows0, rows1)
    AV = (av0, av1)
    DV = (dv0, dv1)
    PB = (pb0, pb1)
    SG = (sem_g0, sem_g1)

    def start_gathers(cidx, b):
        k = jnp.minimum(cidx, NCHUNK - 1)
        pltpu.async_copy(asrc_hbm.at[src2d.at[k]], AV[b], SG[b])
        pltpu.async_copy(adst_hbm.at[dst2d.at[k]], DV[b], SG[b])
        pltpu.async_copy(x_hbm.at[src2d.at[k]], RW[b], SG[b])

    def drain_gathers(cidx, b):
        k = jnp.minimum(cidx, NCHUNK - 1)
        pltpu.make_async_copy(asrc_hbm.at[src2d.at[k]], AV[b], SG[b]).wait()
        pltpu.make_async_copy(adst_hbm.at[dst2d.at[k]], DV[b], SG[b]).wait()
        pltpu.make_async_copy(x_hbm.at[src2d.at[k]], RW[b], SG[b]).wait()

    def compute(b):
        for i in range(CHUNK // 16):
            sl = pl.ds(i * 16, 16)
            PB[b][sl] = jnp.exp(_sigmoid(AV[b][sl] + DV[b][sl]))
        _scale_rows(PB[b], RW[b], D_IN // 16)

    def sync_writes(cidx, b):
        off = base + cidx * CHUNK
        pltpu.sync_copy(PB[b], p_hbm.at[pl.ds(off, CHUNK)])
        pltpu.sync_copy(PB[b], s_sh.at[dst2d.at[cidx]], add=True)
        pltpu.sync_copy(RW[b], g1_sh.at[dst2d.at[cidx]], add=True)

    start_gathers(0, 0)
    drain_gathers(0, 0)
    start_gathers(1, 1)
    compute(0)
    sync_writes(0, 0)

    def step(cidx, b):
        drain_gathers(cidx, b)
        start_gathers(cidx + 1, 1 - b)
        compute(b)
        sync_writes(cidx, b)

    def pair(m, carry):
        step(2 * m + 1, 1)
        step(2 * m + 2, 0)
        return carry

    lax.fori_loop(0, (NCHUNK - 1) // 2, pair, 0)
    drain_gathers(NCHUNK - 1, 1)
    plsc.subcore_barrier()
    pltpu.sync_copy(g1_sh.at[pl.ds(sub * RPS, RPS)],
                    g1p_hbm.at[c, pl.ds(sub * RPS, RPS)])
    pltpu.sync_copy(s_sh.at[pl.ds(sub * RPS, RPS)],
                    sp_hbm.at[c, pl.ds(sub * RPS, RPS)])


def _sc_pass3_body(x2_hbm, p2_hbm, src2_hbm, dst2_hbm, zsm_hbm, g3p_hbm,
                   src2d, dst2d, p2d, rows0, rows1,
                   g3_sh, sem_g0, sem_g1):
    c = lax.axis_index("c")
    sub = lax.axis_index("s")
    wid = sub * NC + c
    pltpu.sync_copy(zsm_hbm, g3_sh.at[pl.ds(sub * RPS, RPS)])
    pltpu.sync_copy(src2_hbm.at[pl.ds(wid * NCHUNK, NCHUNK)], src2d)
    pltpu.sync_copy(dst2_hbm.at[pl.ds(wid * NCHUNK, NCHUNK)], dst2d)
    pltpu.sync_copy(p2_hbm.at[pl.ds(wid * NCHUNK, NCHUNK)], p2d)
    plsc.subcore_barrier()

    RW = (rows0, rows1)
    SG = (sem_g0, sem_g1)

    def start_gathers(cidx, b):
        k = jnp.minimum(cidx, NCHUNK - 1)
        pltpu.async_copy(x2_hbm.at[src2d.at[k]], RW[b], SG[b])

    def drain_gathers(cidx, b):
        k = jnp.minimum(cidx, NCHUNK - 1)
        pltpu.make_async_copy(x2_hbm.at[src2d.at[k]], RW[b], SG[b]).wait()

    def compute(cidx, b):
        _scale_rows(p2d.at[cidx], RW[b], 2)

    def sync_writes(cidx, b):
        pltpu.sync_copy(RW[b], g3_sh.at[dst2d.at[cidx]], add=True)

    start_gathers(0, 0)
    drain_gathers(0, 0)
    start_gathers(1, 1)
    compute(0, 0)
    sync_writes(0, 0)

    def step(cidx, b):
        drain_gathers(cidx, b)
        start_gathers(cidx + 1, 1 - b)
        compute(cidx, b)
        sync_writes(cidx, b)

    def pair(m, carry):
        step(2 * m + 1, 1)
        step(2 * m + 2, 0)
        return carry

    lax.fori_loop(0, (NCHUNK - 1) // 2, pair, 0)
    drain_gathers(NCHUNK - 1, 1)
    plsc.subcore_barrier()
    pltpu.sync_copy(g3_sh.at[pl.ds(sub * RPS, RPS)],
                    g3p_hbm.at[c, pl.ds(sub * RPS, RPS)])


def _tc_attn_body(x_ref, w1_ref, att2_ref, a2_ref):
    w12 = lax.dot_general(w1_ref[...], att2_ref[...], (((1,), (0,)), ((), ())),
                          preferred_element_type=jnp.float32)
    a2_ref[...] = lax.dot_general(w12, x_ref[...], (((0,), (1,)), ((), ())),
                                  preferred_element_type=jnp.float32)


def _elu(v):
    return jnp.where(v > 0.0, v, jnp.exp(jnp.minimum(v, 0.0)) - 1.0)


def _tc_mid_body(g1p_ref, sp_ref, w1_ref, w2_ref, x2_ref, x2p_ref, *, blk):
    s = sp_ref[:, 0] + sp_ref[:, 1] + 1e-16
    g1 = (g1p_ref[0] + g1p_ref[1]) / s[:, None]
    x1 = _elu(jnp.dot(g1, w1_ref[...], preferred_element_type=jnp.float32))
    x2 = jnp.dot(x1, w2_ref[...], preferred_element_type=jnp.float32)
    x2_ref[...] = x2
    x2p_ref[...] = jnp.concatenate(
        [x2, jnp.zeros((blk, 32 - D_LAT), jnp.float32)], axis=1)


def _tc_final_body(g3p_ref, sp_ref, w1_ref, w2_ref, x4_ref, *, blk):
    s = sp_ref[:, 0] + sp_ref[:, 1] + 1e-16
    g3 = (g3p_ref[0] + g3p_ref[1])[:, :D_LAT] / s[:, None]
    x3 = _elu(lax.dot_general(g3, w2_ref[...], (((1,), (1,)), ((), ())),
                              preferred_element_type=jnp.float32))
    x4_ref[...] = lax.dot_general(x3, w1_ref[...], (((1,), (1,)), ((), ())),
                                  preferred_element_type=jnp.float32)


@jax.jit
def kernel(x, edge_index, W1, att_src, att_dst, W2):
    src2 = edge_index[0].reshape(E // CHUNK, CHUNK)
    dst2 = edge_index[1].reshape(E // CHUNK, CHUNK)
    att2 = jnp.stack([att_src, att_dst], axis=1)

    a2 = pl.pallas_call(
        _tc_attn_body,
        out_shape=jax.ShapeDtypeStruct((2, N), jnp.float32),
    )(x, W1, att2)

    mesh = plsc.VectorSubcoreMesh(core_axis_name="c", subcore_axis_name="s")
    sc_params = pltpu.CompilerParams(needs_layout_passes=False,
                                     use_tc_tiling_on_sc=False)
    zbig = jnp.zeros((RPS, D_IN), jnp.float32)
    zvec = jnp.zeros((RPS,), jnp.float32)
    zsm = jnp.zeros((RPS, 32), jnp.float32)

    sc1 = pl.kernel(
        _sc_pass1_body,
        out_type=[
            jax.ShapeDtypeStruct((NC, NPAD, D_IN), jnp.float32),
            jax.ShapeDtypeStruct((NC, NPAD), jnp.float32),
            jax.ShapeDtypeStruct((E,), jnp.float32),
        ],
        mesh=mesh,
        scratch_types=(
            [pltpu.VMEM((NCHUNK, CHUNK), jnp.int32),
             pltpu.VMEM((NCHUNK, CHUNK), jnp.int32)]
            + [pltpu.VMEM((CHUNK, D_IN), jnp.float32),
               pltpu.VMEM((CHUNK,), jnp.float32),
               pltpu.VMEM((CHUNK,), jnp.float32),
               pltpu.VMEM((CHUNK,), jnp.float32)] * 2
            + [pltpu.VMEM_SHARED((NPAD, D_IN), jnp.float32),
               pltpu.VMEM_SHARED((NPAD,), jnp.float32),
               pltpu.SemaphoreType.DMA,
               pltpu.SemaphoreType.DMA]
        ),
        compiler_params=sc_params,
    )
    g1p, sp, p = sc1(x, a2[0], a2[1], src2, dst2, zbig, zvec)
    sp_t = sp.T

    blk = 1000
    x2, x2p = pl.pallas_call(
        functools.partial(_tc_mid_body, blk=blk),
        grid=(N // blk,),
        in_specs=[
            pl.BlockSpec((NC, blk, D_IN), lambda i: (0, i, 0)),
            pl.BlockSpec((blk, NC), lambda i: (i, 0)),
            pl.BlockSpec((D_IN, D_HID), lambda i: (0, 0)),
            pl.BlockSpec((D_HID, D_LAT), lambda i: (0, 0)),
        ],
        out_specs=[
            pl.BlockSpec((blk, D_LAT), lambda i: (i, 0)),
            pl.BlockSpec((blk, 32), lambda i: (i, 0)),
        ],
        out_shape=[
            jax.ShapeDtypeStruct((N, D_LAT), jnp.float32),
            jax.ShapeDtypeStruct((N, 32), jnp.float32),
        ],
    )(g1p, sp_t, W1, W2)

    sc3 = pl.kernel(
        _sc_pass3_body,
        out_type=jax.ShapeDtypeStruct((NC, NPAD, 32), jnp.float32),
        mesh=mesh,
        scratch_types=(
            [pltpu.VMEM((NCHUNK, CHUNK), jnp.int32),
             pltpu.VMEM((NCHUNK, CHUNK), jnp.int32),
             pltpu.VMEM((NCHUNK, CHUNK), jnp.float32),
             pltpu.VMEM((CHUNK, 32), jnp.float32),
             pltpu.VMEM((CHUNK, 32), jnp.float32)]
            + [pltpu.VMEM_SHARED((NPAD, 32), jnp.float32),
               pltpu.SemaphoreType.DMA,
               pltpu.SemaphoreType.DMA]
        ),
        compiler_params=sc_params,
    )
    g3p = sc3(x2p, p.reshape(E // CHUNK, CHUNK), src2, dst2, zsm)

    x4 = pl.pallas_call(
        functools.partial(_tc_final_body, blk=blk),
        grid=(N // blk,),
        in_specs=[
            pl.BlockSpec((NC, blk, 32), lambda i: (0, i, 0)),
            pl.BlockSpec((blk, NC), lambda i: (i, 0)),
            pl.BlockSpec((D_IN, D_HID), lambda i: (0, 0)),
            pl.BlockSpec((D_HID, D_LAT), lambda i: (0, 0)),
        ],
        out_specs=pl.BlockSpec((blk, D_IN), lambda i: (i, 0)),
        out_shape=jax.ShapeDtypeStruct((N, D_IN), jnp.float32),
    )(g3p, sp_t, W1, W2)

    return (x2, x4)
